# baseline (speedup 1.0000x reference)
import functools

import jax
import jax.numpy as jnp
from jax import lax
from jax.experimental import pallas as pl
from jax.experimental.pallas import tpu as pltpu
from jax.experimental.pallas import tpu_sc as plsc

B = 16384
D = 32
L = 16

_info = plsc.get_sparse_core_info()
NC, NS = _info.num_cores, _info.num_subcores
NW = NC * NS
BPW = B // NW
GROUPS = BPW // L


def _sc_kernel(uid_hbm, iid_hbm, ut_hbm, it_hbm, ub_hbm, ib_hbm, gb_hbm,
               out_hbm,
               uidx_v, iidx_v, urows_v, irows_v, ub_v, ib_v, gb_v, out_v,
               sem_u, sem_i, sem_ub, sem_ib):
    wid = lax.axis_index("s") * NC + lax.axis_index("c")
    base = wid * BPW

    pltpu.sync_copy(uid_hbm.at[pl.ds(base, BPW)], uidx_v)
    pltpu.sync_copy(iid_hbm.at[pl.ds(base, BPW)], iidx_v)
    pltpu.sync_copy(gb_hbm, gb_v)

    cp_u = pltpu.make_async_copy(ut_hbm.at[uidx_v], urows_v, sem_u)
    cp_i = pltpu.make_async_copy(it_hbm.at[iidx_v], irows_v, sem_i)
    cp_ub = pltpu.make_async_copy(ub_hbm.at[uidx_v], ub_v, sem_ub)
    cp_ib = pltpu.make_async_copy(ib_hbm.at[iidx_v], ib_v, sem_ib)
    cp_u.start()
    cp_i.start()
    cp_ub.start()
    cp_ib.start()
    cp_u.wait()
    cp_i.wait()
    cp_ub.wait()
    cp_ib.wait()

    lanes = lax.iota(jnp.int32, L)
    gb = gb_v[...]

    def group_body(g, carry):
        row_idx = g * L + lanes
        acc = ub_v[pl.ds(g * L, L)] + ib_v[pl.ds(g * L, L)] + gb
        for d in range(D):
            col = jnp.full((L,), d, jnp.int32)
            cu = plsc.load_gather(urows_v, [row_idx, col])
            ci = plsc.load_gather(irows_v, [row_idx, col])
            acc = acc + cu * ci
        out_v[pl.ds(g * L, L)] = acc
        return carry

    lax.fori_loop(0, GROUPS, group_body, 0)
    pltpu.sync_copy(out_v, out_hbm.at[pl.ds(base, BPW)])


@jax.jit
def _run(user_ids, item_ids, user_table, item_table, ub_flat, ib_flat,
         global_bias):
    mesh = plsc.VectorSubcoreMesh(core_axis_name="c", subcore_axis_name="s")
    f = functools.partial(
        pl.kernel,
        mesh=mesh,
        compiler_params=pltpu.CompilerParams(
            needs_layout_passes=False, use_tc_tiling_on_sc=False),
        out_type=jax.ShapeDtypeStruct((B,), jnp.float32),
        scratch_types=[
            pltpu.VMEM((BPW,), jnp.int32),
            pltpu.VMEM((BPW,), jnp.int32),
            pltpu.VMEM((BPW, D), jnp.float32),
            pltpu.VMEM((BPW, D), jnp.float32),
            pltpu.VMEM((BPW,), jnp.float32),
            pltpu.VMEM((BPW,), jnp.float32),
            pltpu.VMEM((L,), jnp.float32),
            pltpu.VMEM((BPW,), jnp.float32),
            pltpu.SemaphoreType.DMA,
            pltpu.SemaphoreType.DMA,
            pltpu.SemaphoreType.DMA,
            pltpu.SemaphoreType.DMA,
        ],
    )(_sc_kernel)
    return f(user_ids, item_ids, user_table, item_table, ub_flat, ib_flat,
             global_bias)


def kernel(user_ids, item_ids, user_table, item_table, user_bias, item_bias,
           global_bias):
    return _run(user_ids, item_ids, user_table, item_table,
                user_bias.reshape(-1), item_bias.reshape(-1),
                jnp.broadcast_to(global_bias, (L,)))
